# baseline (speedup 1.0000x reference)
import functools

import jax
import jax.numpy as jnp
from jax.experimental import pallas as pl
from jax.experimental.pallas import tpu as pltpu


@functools.lru_cache(maxsize=None)
def _make_row_gather(V: int, L: int):
    NB = L // 128

    def body(ids_ref, cooc_ref, out_ref, ids_smem, sem, sem2):
        pltpu.make_async_copy(ids_ref, ids_smem, sem2).start()
        pltpu.make_async_copy(ids_ref, ids_smem, sem2).wait()
        tok = ids_smem[0, 127]
        pltpu.make_async_copy(
            cooc_ref.at[pl.ds(tok, 1)], out_ref.at[0], sem
        ).start()
        pltpu.make_async_copy(
            cooc_ref.at[pl.ds(tok, 1)], out_ref.at[0], sem
        ).wait()

    return pl.pallas_call(
        body,
        grid=(1,),
        in_specs=[
            pl.BlockSpec((1, 128), lambda i: (0, NB - 1),
                         memory_space=pltpu.MemorySpace.VMEM),
            pl.BlockSpec(memory_space=pltpu.MemorySpace.HBM),
        ],
        out_specs=pl.BlockSpec(memory_space=pltpu.MemorySpace.HBM),
        scratch_shapes=[
            pltpu.SMEM((1, 128), jnp.int32),
            pltpu.SemaphoreType.DMA,
            pltpu.SemaphoreType.DMA,
        ],
        out_shape=jax.ShapeDtypeStruct((1, 1, V), jnp.float32),
    )


def kernel(_, decoder_input_ids, cooc):
    V = cooc.shape[0]
    L = decoder_input_ids.shape[1]
    ids = decoder_input_ids.astype(jnp.int32)
    return _make_row_gather(V, L)(ids, cooc)


# speedup vs baseline: 1.0610x; 1.0610x over previous
import functools

import jax
import jax.numpy as jnp
from jax import lax
from jax.experimental import pallas as pl
from jax.experimental.pallas import tpu as pltpu


@functools.lru_cache(maxsize=None)
def _make_row_gather(V: int, L: int):
    NB = L // 128

    def body(ids_ref, cooc_ref, out_ref, sem):
        ids = ids_ref[...]
        lane = lax.broadcasted_iota(jnp.int32, (1, 128), 1)
        tok = jnp.max(jnp.where(lane == 127, ids, 0))
        pltpu.make_async_copy(
            cooc_ref.at[pl.ds(tok, 1)], out_ref.at[0], sem
        ).start()
        pltpu.make_async_copy(
            cooc_ref.at[pl.ds(tok, 1)], out_ref.at[0], sem
        ).wait()

    return pl.pallas_call(
        body,
        grid=(1,),
        in_specs=[
            pl.BlockSpec((1, 128), lambda i: (0, NB - 1),
                         memory_space=pltpu.MemorySpace.VMEM),
            pl.BlockSpec(memory_space=pltpu.MemorySpace.HBM),
        ],
        out_specs=pl.BlockSpec(memory_space=pltpu.MemorySpace.HBM),
        scratch_shapes=[pltpu.SemaphoreType.DMA],
        out_shape=jax.ShapeDtypeStruct((1, 1, V), jnp.float32),
    )


def kernel(_, decoder_input_ids, cooc):
    V = cooc.shape[0]
    L = decoder_input_ids.shape[1]
    ids = decoder_input_ids.astype(jnp.int32)
    return _make_row_gather(V, L)(ids, cooc)


# speedup vs baseline: 1.0878x; 1.0253x over previous
import functools

import jax
import jax.numpy as jnp
from jax.experimental import pallas as pl
from jax.experimental.pallas import tpu as pltpu


@functools.lru_cache(maxsize=None)
def _make_row_gather(V: int, L: int):
    NB = L // 128

    def body(ids_ref, cooc_ref, out_ref, sem):
        tok = ids_ref[0, 127]
        pltpu.make_async_copy(
            cooc_ref.at[pl.ds(tok, 1)], out_ref.at[0], sem
        ).start()
        pltpu.make_async_copy(
            cooc_ref.at[pl.ds(tok, 1)], out_ref.at[0], sem
        ).wait()

    return pl.pallas_call(
        body,
        grid=(1,),
        in_specs=[
            pl.BlockSpec((1, 128), lambda i: (0, NB - 1),
                         memory_space=pltpu.MemorySpace.SMEM),
            pl.BlockSpec(memory_space=pltpu.MemorySpace.HBM),
        ],
        out_specs=pl.BlockSpec(memory_space=pltpu.MemorySpace.HBM),
        scratch_shapes=[pltpu.SemaphoreType.DMA],
        out_shape=jax.ShapeDtypeStruct((1, 1, V), jnp.float32),
    )


def kernel(_, decoder_input_ids, cooc):
    V = cooc.shape[0]
    L = decoder_input_ids.shape[1]
    ids = decoder_input_ids.astype(jnp.int32)
    return _make_row_gather(V, L)(ids, cooc)
